# baseline (speedup 1.0000x reference)
import functools

import jax
import jax.numpy as jnp
from jax.experimental import pallas as pl
from jax.experimental.pallas import tpu as pltpu

T = 2048
D = 1024
F = 4096
E = 8
TOP_K = 2

BT = 256
BF = 512


def _first_true(mask):
    c = jax.lax.broadcasted_iota(jnp.int32, mask.shape, 1)
    first = jnp.min(jnp.where(mask, c, mask.shape[1]), axis=1, keepdims=True)
    return c == first


def _dense_kernel(x_ref, r_ref, w1_ref, w3_ref, w2_ref, o_ref, cw_ref):
    e = pl.program_id(1)
    j = pl.program_id(2)

    @pl.when(jnp.logical_and(e == 0, j == 0))
    def _compute_routing():
        logits = r_ref[...]
        m = jnp.max(logits, axis=1, keepdims=True)
        p = jnp.exp(logits - m)
        p = p / jnp.sum(p, axis=1, keepdims=True)
        m1 = jnp.max(p, axis=1, keepdims=True)
        is1 = _first_true(p == m1)
        p_wo1 = jnp.where(is1, -jnp.inf, p)
        m2 = jnp.max(p_wo1, axis=1, keepdims=True)
        is2 = _first_true((p_wo1 == m2))
        denom = m1 + m2
        cw_ref[...] = jnp.where(is1, m1 / denom, 0.0) + jnp.where(is2, m2 / denom, 0.0)

    @pl.when(jnp.logical_and(e == 0, j == 0))
    def _init_out():
        o_ref[...] = jnp.zeros_like(o_ref)

    x = x_ref[...]
    w1 = w1_ref[0]
    w3 = w3_ref[0]
    w2 = w2_ref[0]
    dn = (((1,), (1,)), ((), ()))
    h1 = jax.lax.dot_general(x, w1, dn, preferred_element_type=jnp.float32)
    h3 = jax.lax.dot_general(x, w3, dn, preferred_element_type=jnp.float32)
    act = h1 * jax.nn.sigmoid(h1) * h3
    oe = jax.lax.dot_general(act, w2, dn, preferred_element_type=jnp.float32)
    cw = cw_ref[...]
    sel = jax.lax.broadcasted_iota(jnp.int32, cw.shape, 1) == e
    cwcol = jnp.sum(jnp.where(sel, cw, 0.0), axis=1, keepdims=True)
    o_ref[...] += cwcol * oe


@jax.jit
def kernel(hidden_states, router_logits, w1, w2, w3):
    grid = (T // BT, E, F // BF)
    return pl.pallas_call(
        _dense_kernel,
        grid=grid,
        in_specs=[
            pl.BlockSpec((BT, D), lambda i, e, j: (i, 0)),
            pl.BlockSpec((BT, E), lambda i, e, j: (i, 0)),
            pl.BlockSpec((1, BF, D), lambda i, e, j: (e, j, 0)),
            pl.BlockSpec((1, BF, D), lambda i, e, j: (e, j, 0)),
            pl.BlockSpec((1, D, BF), lambda i, e, j: (e, 0, j)),
        ],
        out_specs=pl.BlockSpec((BT, D), lambda i, e, j: (i, 0)),
        out_shape=jax.ShapeDtypeStruct((T, D), jnp.float32),
        scratch_shapes=[pltpu.VMEM((BT, E), jnp.float32)],
    )(hidden_states, router_logits, w1, w3, w2)
